# baseline (speedup 1.0000x reference)
import jax
import jax.numpy as jnp
from jax.experimental import pallas as pl
from jax.experimental.pallas import tpu as pltpu

N_CODES = 1024
DIM = 128
H = 32
W = 32
HW = H * W
BATCH = 16


NG = 8


def _mk_in(z_hbm, z_buf, sems, bb, slot, k):
    return pltpu.make_async_copy(
        z_hbm.at[bb, :, k, :],
        z_buf.at[slot, :, pl.ds(k * 128, 128)],
        sems.at[slot],
    )


def _mk_out(q_buf, q_hbm, sems, bb, slot, k):
    return pltpu.make_async_copy(
        q_buf.at[slot, :, pl.ds(k * 128, 128)],
        q_hbm.at[bb, :, k, :],
        sems.at[slot],
    )


def _vq_body(z_hbm, e_ref, e2_ref, ecat_ref, iota_ref,
             q_hbm, idx_ref, loss_ref,
             z_buf, q_buf, in_sems, out_sems):
    b = pl.program_id(0)
    slot = jax.lax.rem(b, 2)

    @pl.when(b == 0)
    def _prologue():
        for k in range(NG):
            _mk_in(z_hbm, z_buf, in_sems, 0, 0, k).start()

    @pl.when(b + 1 < BATCH)
    def _prefetch():
        nxt = jax.lax.rem(b + 1, 2)
        for k in range(NG):
            _mk_in(z_hbm, z_buf, in_sems, b + 1, nxt, k).start()

    for k in range(NG):
        _mk_in(z_hbm, z_buf, in_sems, b, slot, k).wait()

    z = z_buf[slot]
    e = e_ref[...]

    in_norm = jnp.sum(z * z, axis=0, keepdims=True)
    e_norm = jnp.sum(e * e, axis=1, keepdims=True)
    dot2_t = jax.lax.dot_general(
        e2_ref[...], z, (((1,), (0,)), ((), ())))
    d = (in_norm + e_norm) - dot2_t

    d_min = jnp.min(d, axis=0, keepdims=True)
    code_iota = iota_ref[...]
    masked = jnp.where(d == d_min, code_iota, float(N_CODES))
    idx_f = jnp.min(masked, axis=0, keepdims=True)

    onehot = (code_iota == idx_f).astype(jnp.bfloat16)
    qq = jax.lax.dot_general(
        ecat_ref[...], onehot, (((0,), (0,)), ((), ())),
        preferred_element_type=jnp.float32)
    q_t = qq[:DIM, :] + qq[DIM:, :]

    diff = q_t - z

    @pl.when(b >= 2)
    def _drain_prev():
        for k in range(NG):
            _mk_out(q_buf, q_hbm, out_sems, b - 2, slot, k).wait()

    q_buf[slot] = z + diff
    for k in range(NG):
        _mk_out(q_buf, q_hbm, out_sems, b, slot, k).start()

    idx_ref[0] = idx_f

    @pl.when(b == 0)
    def _init():
        loss_ref[...] = jnp.zeros((1, 1), jnp.float32)

    loss_ref[...] += jnp.sum(diff * diff, keepdims=True)

    @pl.when(b == BATCH - 1)
    def _epilogue():
        for k in range(NG):
            _mk_out(q_buf, q_hbm, out_sems, b - 1, jax.lax.rem(b + 1, 2), k).wait()
        for k in range(NG):
            _mk_out(q_buf, q_hbm, out_sems, b, slot, k).wait()


def kernel(z_e, embed_w):
    z8 = z_e.reshape(BATCH, DIM, 8, 128)
    iota_col = jnp.arange(N_CODES, dtype=jnp.float32).reshape(N_CODES, 1)
    e_hi = embed_w.astype(jnp.bfloat16)
    e_lo = (embed_w - e_hi.astype(jnp.float32)).astype(jnp.bfloat16)
    e_cat = jnp.concatenate([e_hi, e_lo], axis=1)
    e2 = embed_w + embed_w
    q4, idx_f, loss = pl.pallas_call(
        _vq_body,
        grid=(BATCH,),
        in_specs=[
            pl.BlockSpec(memory_space=pl.ANY),
            pl.BlockSpec((N_CODES, DIM), lambda b: (0, 0)),
            pl.BlockSpec((N_CODES, DIM), lambda b: (0, 0)),
            pl.BlockSpec((N_CODES, 2 * DIM), lambda b: (0, 0)),
            pl.BlockSpec((N_CODES, 1), lambda b: (0, 0)),
        ],
        out_specs=[
            pl.BlockSpec(memory_space=pl.ANY),
            pl.BlockSpec((1, 1, HW), lambda b: (b, 0, 0)),
            pl.BlockSpec((1, 1), lambda b: (0, 0)),
        ],
        out_shape=[
            jax.ShapeDtypeStruct((BATCH, DIM, 8, 128), jnp.float32),
            jax.ShapeDtypeStruct((BATCH, 1, HW), jnp.float32),
            jax.ShapeDtypeStruct((1, 1), jnp.float32),
        ],
        scratch_shapes=[
            pltpu.VMEM((2, DIM, HW), jnp.float32),
            pltpu.VMEM((2, DIM, HW), jnp.float32),
            pltpu.SemaphoreType.DMA((2,)),
            pltpu.SemaphoreType.DMA((2,)),
        ],
    )(z8, embed_w, e2, e_cat, iota_col)
    q4 = q4.reshape(BATCH, DIM, H, W)
    indices = idx_f.reshape(BATCH, HW).astype(jnp.int32)
    n_elems = BATCH * DIM * HW
    commitment = (loss[0, 0] / n_elems) * 0.25
    return (q4, indices, commitment)
